# baseline (speedup 1.0000x reference)
import functools

import jax
import jax.numpy as jnp
from jax import lax
from jax.experimental import pallas as pl
from jax.experimental.pallas import tpu as pltpu
from jax.experimental.pallas import tpu_sc as plsc

NC, NS, LANES = 2, 16, 16
NW = NC * NS

B, TL, AL, D = 4096, 200, 20, 64
TLP = 208
ALP = 32
ALG = 24
BPW = B // NW
DC = D // LANES


def _sc_features(tflat, aflat, table, atable):
  mesh = plsc.VectorSubcoreMesh(
      core_axis_name="c", subcore_axis_name="s", num_cores=NC, num_subcores=NS)

  @functools.partial(
      pl.kernel,
      out_type=jax.ShapeDtypeStruct((B, 2 * D), jnp.float32),
      mesh=mesh,
      scratch_types=[
          pltpu.VMEM((BPW * TLP,), jnp.int32),
          pltpu.VMEM((BPW * ALP,), jnp.int32),
          pltpu.VMEM((TLP, D), jnp.float32),
          pltpu.VMEM((ALG, D), jnp.float32),
          pltpu.VMEM((BPW, 2 * D), jnp.float32),
          pltpu.SemaphoreType.DMA,
      ],
      compiler_params=pltpu.CompilerParams(
          use_tc_tiling_on_sc=False, needs_layout_passes=False),
  )
  def k(tidx_hbm, aidx_hbm, tab_hbm, atab_hbm, out_hbm,
        idxt, idxa, rows_t, rows_a, outb, sem):
    wid = lax.axis_index("s") * NC + lax.axis_index("c")
    base = wid * BPW
    zi = jnp.zeros((LANES,), jnp.int32)
    zf = jnp.zeros((LANES,), jnp.float32)

    pltpu.sync_copy(tidx_hbm.at[pl.ds(base * TLP, BPW * TLP)], idxt)
    pltpu.sync_copy(aidx_hbm.at[pl.ds(base * ALP, BPW * ALP)], idxa)

    def row(b, carry):
      ot = pl.multiple_of(b * TLP, TLP)
      oa = pl.multiple_of(b * ALP, ALP)
      cp0 = pltpu.async_copy(tab_hbm.at[idxt.at[pl.ds(ot, 104)]],
                             rows_t.at[pl.ds(0, 104)], sem)
      cp1 = pltpu.async_copy(tab_hbm.at[idxt.at[pl.ds(ot + 104, 104)]],
                             rows_t.at[pl.ds(104, 104)], sem)
      cp2 = pltpu.async_copy(atab_hbm.at[idxa.at[pl.ds(oa, ALG)]],
                             rows_a, sem)
      cp0.wait()
      cp1.wait()
      cp2.wait()

      lt = zi
      for c in range(TLP // LANES):
        lt = lt + plsc.all_reduce_population_count(
            idxt[pl.ds(ot + c * LANES, LANES)] != 0)
      la = zi
      for c in range(ALP // LANES):
        la = la + plsc.all_reduce_population_count(
            idxa[pl.ds(oa + c * LANES, LANES)] != 0)

      acc_t = [zf] * DC
      for p in range(TL):
        m = lt > p
        for d in range(DC):
          v = rows_t[p, pl.ds(d * LANES, LANES)]
          acc_t[d] = acc_t[d] + jnp.where(m, v, 0.0)
      acc_a = [zf] * DC
      for p in range(AL):
        m = la > p
        for d in range(DC):
          v = rows_a[p, pl.ds(d * LANES, LANES)]
          acc_a[d] = acc_a[d] + jnp.where(m, v, 0.0)

      inv_t = 1.0 / jnp.maximum(lt.astype(jnp.float32), 1.0)
      inv_a = 1.0 / jnp.maximum(la.astype(jnp.float32), 1.0)
      for d in range(DC):
        outb[b, pl.ds(d * LANES, LANES)] = acc_t[d] * inv_t
        outb[b, pl.ds(D + d * LANES, LANES)] = acc_a[d] * inv_a
      return carry

    lax.fori_loop(0, BPW, row, 0)
    pltpu.sync_copy(outb, out_hbm.at[pl.ds(base, BPW)])

  return k(tflat, aflat, table, atable)


def _tc_logits(feat, w, bias):
  def body(f_ref, w_ref, b_ref, o_ref):
    o_ref[...] = lax.dot_general(
        f_ref[...], w_ref[...], (((1,), (1,)), ((), ())),
        preferred_element_type=jnp.float32,
        precision=lax.Precision.HIGHEST) + b_ref[...]

  return pl.pallas_call(
      body,
      out_shape=jax.ShapeDtypeStruct((B, w.shape[0]), jnp.float32),
  )(feat, w, bias.reshape(1, -1))


def kernel(text_raw_indices, aspect_indices, embedding_matrix,
           aspect_embedding_matrix, W, b):
  tflat = jnp.pad(text_raw_indices.astype(jnp.int32),
                  ((0, 0), (0, TLP - TL))).reshape(-1)
  aflat = jnp.pad(aspect_indices.astype(jnp.int32),
                  ((0, 0), (0, ALP - AL))).reshape(-1)
  feat = _sc_features(tflat, aflat, embedding_matrix, aspect_embedding_matrix)
  return _tc_logits(feat, W, b)
